# Initial kernel scaffold; baseline (speedup 1.0000x reference)
import jax
import jax.numpy as jnp
from jax.experimental import pallas as pl


def kernel(input, table):
    raise NotImplementedError("write your pallas kernel here")


# speedup vs baseline: 1.0223x; 1.0223x over previous
import functools

import jax
import jax.numpy as jnp
from jax import lax
from jax.experimental import pallas as pl
from jax.experimental.pallas import tpu as pltpu
from jax.experimental.pallas import tpu_sc as plsc

CHUNK = 128


@functools.cache
def _build(n_total: int, emb_dim: int):
    info = plsc.get_sparse_core_info()
    nc, ns = info.num_cores, info.num_subcores
    nw = nc * ns
    assert n_total % (nw * CHUNK) == 0
    n_chunks = n_total // (nw * CHUNK)

    mesh = plsc.VectorSubcoreMesh(core_axis_name="c", subcore_axis_name="s")

    def body(idx_hbm, table_hbm, out_hbm, idx_v, rows_v, gsem):
        wid = lax.axis_index("s") * nc + lax.axis_index("c")
        base = wid * (n_chunks * CHUNK)
        pltpu.sync_copy(idx_hbm.at[wid], idx_v)

        def step(j, carry):
            pltpu.async_copy(table_hbm.at[idx_v.at[j]], rows_v, gsem).wait()
            pltpu.sync_copy(rows_v, out_hbm.at[pl.ds(base + j * CHUNK, CHUNK)])
            return carry

        lax.fori_loop(0, n_chunks, step, 0, unroll=False)

    return pl.kernel(
        body,
        out_type=jax.ShapeDtypeStruct((n_total, emb_dim), jnp.float32),
        mesh=mesh,
        compiler_params=pltpu.CompilerParams(use_tc_tiling_on_sc=False),
        scratch_types=[
            pltpu.VMEM((n_chunks, CHUNK), jnp.int32),
            pltpu.VMEM((CHUNK, emb_dim), jnp.float32),
            pltpu.SemaphoreType.DMA,
        ],
    ), nw, n_chunks


def kernel(input, table):
    batch, hist = input.shape
    _, emb_dim = table.shape
    n_total = batch * hist
    k, nw, n_chunks = _build(n_total, emb_dim)
    idx = input.reshape(nw, n_chunks, CHUNK).astype(jnp.int32)
    out = k(idx, table)
    return out.reshape(batch, hist, emb_dim)


# speedup vs baseline: 1.1105x; 1.0863x over previous
import functools

import jax
import jax.numpy as jnp
from jax import lax
from jax.experimental import pallas as pl
from jax.experimental.pallas import tpu as pltpu
from jax.experimental.pallas import tpu_sc as plsc

CHUNK = 512
NBUF = 5


@functools.cache
def _build(n_total: int, emb_dim: int):
    info = plsc.get_sparse_core_info()
    nc, ns = info.num_cores, info.num_subcores
    nw = nc * ns
    assert n_total % (nw * CHUNK) == 0
    n_chunks = n_total // (nw * CHUNK)
    assert n_chunks % NBUF == 0
    n_rounds = n_chunks // NBUF

    mesh = plsc.VectorSubcoreMesh(core_axis_name="c", subcore_axis_name="s")

    def body(idx_hbm, table_hbm, out_hbm, idx_v, rows_v, gsem, wsem):
        wid = lax.axis_index("s") * nc + lax.axis_index("c")
        base = wid * (n_chunks * CHUNK)
        pltpu.sync_copy(idx_hbm.at[wid], idx_v)

        def gather(j, b):
            return pltpu.make_async_copy(
                table_hbm.at[idx_v.at[j]], rows_v.at[b], gsem.at[b]
            )

        def write(j, b):
            return pltpu.make_async_copy(
                rows_v.at[b], out_hbm.at[pl.ds(base + j * CHUNK, CHUNK)], wsem.at[b]
            )

        for b in range(NBUF):
            gather(b, b).start()

        def round_(g, carry):
            j0 = g * NBUF
            for b in range(NBUF):
                gather(j0 + b, b).wait()
                write(j0 + b, b).start()

            @pl.when(g < n_rounds - 1)
            def _refill():
                for b in range(NBUF):
                    write(j0 + b, b).wait()
                    gather(j0 + NBUF + b, b).start()

            return carry

        lax.fori_loop(0, n_rounds, round_, 0, unroll=False)
        for b in range(NBUF):
            write((n_rounds - 1) * NBUF + b, b).wait()

    return pl.kernel(
        body,
        out_type=jax.ShapeDtypeStruct((n_total, emb_dim), jnp.float32),
        mesh=mesh,
        compiler_params=pltpu.CompilerParams(use_tc_tiling_on_sc=False),
        scratch_types=[
            pltpu.VMEM((n_chunks, CHUNK), jnp.int32),
            pltpu.VMEM((NBUF, CHUNK, emb_dim), jnp.float32),
            pltpu.SemaphoreType.DMA((NBUF,)),
            pltpu.SemaphoreType.DMA((NBUF,)),
        ],
    ), nw, n_chunks


def kernel(input, table):
    batch, hist = input.shape
    _, emb_dim = table.shape
    n_total = batch * hist
    k, nw, n_chunks = _build(n_total, emb_dim)
    idx = input.reshape(nw, n_chunks, CHUNK).astype(jnp.int32)
    out = k(idx, table)
    return out.reshape(batch, hist, emb_dim)


# speedup vs baseline: 1.4781x; 1.3310x over previous
import functools

import jax
import jax.numpy as jnp
from jax import lax
from jax.experimental import pallas as pl
from jax.experimental.pallas import tpu as pltpu
from jax.experimental.pallas import tpu_sc as plsc

CHUNK = 128
NBUF = 2
LANES = 16


@functools.cache
def _build(batch: int, hist: int, emb_dim: int, vocab: int):
    info = plsc.get_sparse_core_info()
    nc, ns = info.num_cores, info.num_subcores
    nw = nc * ns
    b_per_w = batch // nw
    assert batch % (nw * CHUNK) == 0
    n_sub = b_per_w // CHUNK
    slice_w = 128
    rows_per_slice = slice_w // emb_dim
    n_chunks = hist * n_sub

    mesh = plsc.VectorSubcoreMesh(core_axis_name="c", subcore_axis_name="s")

    def body(idx_hbm, tab4_hbm, out_hbm, idx_v, idx2_v, buf_v, outv_v, gsem, wsem):
        wid = lax.axis_index("s") * nc + lax.axis_index("c")
        b0 = wid * b_per_w
        pltpu.sync_copy(idx_hbm.at[:, pl.ds(b0, b_per_w)], idx_v)

        def prep(j, b):
            h = j // n_sub
            c = j % n_sub
            for v in range(CHUNK // LANES):
                iv = idx_v[h, pl.ds(c * CHUNK + v * LANES, LANES)]
                idx2_v[b, pl.ds(v * LANES, LANES)] = lax.shift_right_logical(
                    iv, jnp.int32(2)
                )

        def gather(b):
            return pltpu.make_async_copy(
                tab4_hbm.at[idx2_v.at[b]], buf_v.at[b], gsem.at[b]
            )

        def write(j, b):
            h = j // n_sub
            c = j % n_sub
            return pltpu.make_async_copy(
                outv_v.at[b],
                out_hbm.at[h, :, pl.ds(b0 + c * CHUNK, CHUNK)],
                wsem.at[b],
            )

        def transpose(j, b):
            h = j // n_sub
            c = j % n_sub
            buf = buf_v.at[b]
            for g in range(CHUNK // LANES):
                rows = lax.iota(jnp.int32, LANES) + jnp.int32(g * LANES)
                sub = idx_v[h, pl.ds(c * CHUNK + g * LANES, LANES)]
                cbase = (sub & jnp.int32(rows_per_slice - 1)) * jnp.int32(emb_dim)
                for d in range(emb_dim):
                    vals = plsc.load_gather(buf, [rows, cbase + jnp.int32(d)])
                    outv_v[b, d, pl.ds(g * LANES, LANES)] = vals

        for b in range(NBUF):
            prep(b, b)
            gather(b).start()

        def step(j, carry):
            b = lax.rem(j, jnp.int32(NBUF))

            @pl.when(j >= NBUF)
            def _wait_write():
                write(j - NBUF, b).wait()

            gather(b).wait()
            transpose(j, b)

            @pl.when(j + NBUF < n_chunks)
            def _refill():
                prep(j + NBUF, b)
                gather(b).start()

            write(j, b).start()
            return carry

        lax.fori_loop(0, n_chunks, step, 0, unroll=False)
        for j in range(n_chunks - NBUF, n_chunks):
            write(j, j % NBUF).wait()

    return pl.kernel(
        body,
        out_type=jax.ShapeDtypeStruct((hist, emb_dim, batch), jnp.float32),
        mesh=mesh,
        compiler_params=pltpu.CompilerParams(
            use_tc_tiling_on_sc=False, needs_layout_passes=False
        ),
        scratch_types=[
            pltpu.VMEM((hist, b_per_w), jnp.int32),
            pltpu.VMEM((NBUF, CHUNK), jnp.int32),
            pltpu.VMEM((NBUF, CHUNK, slice_w), jnp.float32),
            pltpu.VMEM((NBUF, emb_dim, CHUNK), jnp.float32),
            pltpu.SemaphoreType.DMA((NBUF,)),
            pltpu.SemaphoreType.DMA((NBUF,)),
        ],
    )


def kernel(input, table):
    batch, hist = input.shape
    vocab, emb_dim = table.shape
    k = _build(batch, hist, emb_dim, vocab)
    inp_t = input.T.astype(jnp.int32)
    tab4 = table.reshape(vocab * emb_dim // 128, 128)
    out = k(inp_t, tab4)
    return out.transpose(2, 0, 1)


# speedup vs baseline: 1.5101x; 1.0216x over previous
import functools

import jax
import jax.numpy as jnp
from jax import lax
from jax.experimental import pallas as pl
from jax.experimental.pallas import tpu as pltpu
from jax.experimental.pallas import tpu_sc as plsc

CHUNK = 128
GBUF = 3
OBUF = 2
LANES = 16


@functools.cache
def _build(batch: int, hist: int, emb_dim: int, vocab: int):
    info = plsc.get_sparse_core_info()
    nc, ns = info.num_cores, info.num_subcores
    nw = nc * ns
    b_per_w = batch // nw
    assert batch % (nw * CHUNK) == 0
    n_sub = b_per_w // CHUNK
    n_chunks = hist * n_sub

    mesh = plsc.VectorSubcoreMesh(core_axis_name="c", subcore_axis_name="s")

    def body(idx_hbm, tabp_hbm, out_hbm, idx_v, buf_v, outv_v, gsem, wsem):
        wid = lax.axis_index("s") * nc + lax.axis_index("c")
        b0 = wid * b_per_w
        pltpu.sync_copy(idx_hbm.at[:, pl.ds(b0, b_per_w)], idx_v)

        def gather(j, gb):
            h = j // n_sub
            c = lax.rem(j, n_sub)
            return pltpu.make_async_copy(
                tabp_hbm.at[idx_v.at[h, pl.ds(c * CHUNK, CHUNK)]],
                buf_v.at[gb],
                gsem.at[gb],
            )

        def write(h, ob):
            return pltpu.make_async_copy(
                outv_v.at[ob],
                out_hbm.at[h, :, pl.ds(b0, b_per_w)],
                wsem.at[ob],
            )

        def transpose(j, gb, ob):
            c = lax.rem(j, n_sub)
            buf = buf_v.at[gb]
            for g in range(CHUNK // LANES):
                rows = lax.iota(jnp.int32, LANES) + jnp.int32(g * LANES)
                for d in range(emb_dim):
                    vals = plsc.load_gather(
                        buf, [rows, jnp.full((LANES,), d, jnp.int32)]
                    )
                    outv_v[ob, d, pl.ds(c * CHUNK + g * LANES, LANES)] = vals

        for j in range(GBUF):
            gather(j, j).start()

        def step(j, carry):
            gb = lax.rem(j, jnp.int32(GBUF))
            h = j // n_sub
            c = lax.rem(j, n_sub)
            ob = lax.rem(h, jnp.int32(OBUF))

            @pl.when(jnp.logical_and(c == 0, h >= OBUF))
            def _wait_write():
                write(h - OBUF, ob).wait()

            gather(j, gb).wait()
            transpose(j, gb, ob)

            @pl.when(j + GBUF < n_chunks)
            def _refill():
                gather(j + GBUF, gb).start()

            @pl.when(c == n_sub - 1)
            def _flush():
                write(h, ob).start()

            return carry

        lax.fori_loop(0, n_chunks, step, 0, unroll=False)
        for h in range(hist - OBUF, hist):
            write(h, h % OBUF).wait()

    return pl.kernel(
        body,
        out_type=jax.ShapeDtypeStruct((hist, emb_dim, batch), jnp.float32),
        mesh=mesh,
        compiler_params=pltpu.CompilerParams(
            use_tc_tiling_on_sc=False, needs_layout_passes=False
        ),
        scratch_types=[
            pltpu.VMEM((hist, b_per_w), jnp.int32),
            pltpu.VMEM((GBUF, CHUNK, 128), jnp.float32),
            pltpu.VMEM((OBUF, emb_dim, b_per_w), jnp.float32),
            pltpu.SemaphoreType.DMA((GBUF,)),
            pltpu.SemaphoreType.DMA((OBUF,)),
        ],
    )


def kernel(input, table):
    batch, hist = input.shape
    vocab, emb_dim = table.shape
    k = _build(batch, hist, emb_dim, vocab)
    inp_t = input.T.astype(jnp.int32)
    tabp = jnp.pad(table, ((0, 0), (0, 128 - emb_dim)))
    out = k(inp_t, tabp)
    return out.transpose(2, 0, 1)


# speedup vs baseline: 1.5139x; 1.0025x over previous
import functools

import jax
import jax.numpy as jnp
from jax import lax
from jax.experimental import pallas as pl
from jax.experimental.pallas import tpu as pltpu
from jax.experimental.pallas import tpu_sc as plsc

CHUNK = 128
GBUF = 4
OBUF = 2
LANES = 16
PADW = 128


@functools.cache
def _build(batch: int, hist: int, emb_dim: int, vocab: int):
    info = plsc.get_sparse_core_info()
    nc, ns = info.num_cores, info.num_subcores
    nw = nc * ns
    b_per_w = batch // nw
    assert batch % (nw * CHUNK) == 0
    n_sub = b_per_w // CHUNK
    n_chunks = hist * n_sub
    n_rows = vocab * PADW // emb_dim

    mesh = plsc.VectorSubcoreMesh(core_axis_name="c", subcore_axis_name="s")

    def body(idx_hbm, tab_hbm, out_hbm, idx_v, buf_v, outv_v, gsem, wsem):
        wid = lax.axis_index("s") * nc + lax.axis_index("c")
        b0 = wid * b_per_w
        pltpu.sync_copy(idx_hbm.at[:, pl.ds(b0, b_per_w)], idx_v)

        def gather(j, gb):
            h = j // n_sub
            c = lax.rem(j, n_sub)
            return pltpu.make_async_copy(
                tab_hbm.at[idx_v.at[h, pl.ds(c * CHUNK, CHUNK)]],
                buf_v.at[gb],
                gsem.at[gb],
            )

        def write(h, ob):
            return pltpu.make_async_copy(
                outv_v.at[ob],
                out_hbm.at[h, :, pl.ds(b0, b_per_w)],
                wsem.at[ob],
            )

        def transpose(j, gb, ob):
            c = lax.rem(j, n_sub)
            buf = buf_v.at[gb]
            for g in range(CHUNK // LANES):
                rows = lax.iota(jnp.int32, LANES) + jnp.int32(g * LANES)
                for d in range(emb_dim):
                    vals = plsc.load_gather(
                        buf, [rows, jnp.full((LANES,), d, jnp.int32)]
                    )
                    outv_v[ob, d, pl.ds(c * CHUNK + g * LANES, LANES)] = vals

        for j in range(GBUF):
            gather(j, j).start()

        def step(j, carry):
            gb = lax.rem(j, jnp.int32(GBUF))
            h = j // n_sub
            c = lax.rem(j, n_sub)
            ob = lax.rem(h, jnp.int32(OBUF))

            @pl.when(jnp.logical_and(c == 0, h >= OBUF))
            def _wait_write():
                write(h - OBUF, ob).wait()

            gather(j, gb).wait()
            transpose(j, gb, ob)

            @pl.when(j + GBUF < n_chunks)
            def _refill():
                gather(j + GBUF, gb).start()

            @pl.when(c == n_sub - 1)
            def _flush():
                write(h, ob).start()

            return carry

        lax.fori_loop(0, n_chunks, step, 0, unroll=False)
        for h in range(hist - OBUF, hist):
            write(h, h % OBUF).wait()

    return pl.kernel(
        body,
        out_type=jax.ShapeDtypeStruct((hist, emb_dim, batch), jnp.float32),
        mesh=mesh,
        compiler_params=pltpu.CompilerParams(
            use_tc_tiling_on_sc=False, needs_layout_passes=False
        ),
        scratch_types=[
            pltpu.VMEM((hist, b_per_w), jnp.int32),
            pltpu.VMEM((GBUF, CHUNK, emb_dim), jnp.float32),
            pltpu.VMEM((OBUF, emb_dim, b_per_w), jnp.float32),
            pltpu.SemaphoreType.DMA((GBUF,)),
            pltpu.SemaphoreType.DMA((OBUF,)),
        ],
    )


def kernel(input, table):
    batch, hist = input.shape
    vocab, emb_dim = table.shape
    k = _build(batch, hist, emb_dim, vocab)
    scale = PADW // emb_dim
    inp_t = input.T.astype(jnp.int32) * jnp.int32(scale)
    tabp = jnp.pad(table, ((0, 0), (0, PADW - emb_dim)))
    tab4 = tabp.reshape(vocab * scale, emb_dim)
    out = k(inp_t, tab4)
    return out.transpose(2, 0, 1)


# speedup vs baseline: 2.5234x; 1.6668x over previous
import functools

import jax
import jax.numpy as jnp
from jax import lax
from jax.experimental import pallas as pl
from jax.experimental.pallas import tpu as pltpu
from jax.experimental.pallas import tpu_sc as plsc

CHUNK = 128
GBUF = 8
DEPTH = 4
PADW = 128


def _tc_transpose(inter, batch, hist, emb_dim):
    def body(src_ref, dst_ref):
        dst_ref[...] = src_ref[:, :emb_dim].T[None]

    return pl.pallas_call(
        body,
        grid=(hist,),
        in_specs=[pl.BlockSpec((batch, PADW), lambda h: (h, 0))],
        out_specs=pl.BlockSpec((1, emb_dim, batch), lambda h: (h, 0, 0)),
        out_shape=jax.ShapeDtypeStruct((hist, emb_dim, batch), jnp.float32),
    )(inter)


@functools.cache
def _build(batch: int, hist: int, emb_dim: int, vocab: int):
    info = plsc.get_sparse_core_info()
    nc, ns = info.num_cores, info.num_subcores
    nw = nc * ns
    b_per_w = batch // nw
    assert batch % (nw * CHUNK) == 0
    n_sub = b_per_w // CHUNK
    n_chunks = hist * n_sub

    mesh = plsc.VectorSubcoreMesh(core_axis_name="c", subcore_axis_name="s")

    def body(idx_hbm, tab_hbm, inter_hbm, idx_v, buf_v, gsem, wsem):
        wid = lax.axis_index("s") * nc + lax.axis_index("c")
        b0 = wid * b_per_w
        pltpu.sync_copy(idx_hbm.at[:, pl.ds(b0, b_per_w)], idx_v)

        def gather(j, gb):
            h = j // n_sub
            c = lax.rem(j, n_sub)
            return pltpu.make_async_copy(
                tab_hbm.at[idx_v.at[h, pl.ds(c * CHUNK, CHUNK)]],
                buf_v.at[gb],
                gsem.at[gb],
            )

        def write(j, gb):
            h = j // n_sub
            c = lax.rem(j, n_sub)
            row0 = h * batch + b0 + c * CHUNK
            return pltpu.make_async_copy(
                buf_v.at[gb],
                inter_hbm.at[pl.ds(row0, CHUNK), pl.ds(0, emb_dim)],
                wsem.at[gb],
            )

        for j in range(DEPTH):
            gather(j, j).start()

        def step(j, carry):
            gb = lax.rem(j, jnp.int32(GBUF))
            gbn = lax.rem(j + DEPTH, jnp.int32(GBUF))

            @pl.when(j + DEPTH < n_chunks)
            def _refill():
                @pl.when(j >= GBUF - DEPTH)
                def _drain():
                    write(j + DEPTH - GBUF, gbn).wait()

                gather(j + DEPTH, gbn).start()

            gather(j, gb).wait()
            write(j, gb).start()
            return carry

        lax.fori_loop(0, n_chunks, step, 0, unroll=False)
        for j in range(n_chunks - GBUF, n_chunks):
            write(j, j % GBUF).wait()

    return pl.kernel(
        body,
        out_type=jax.ShapeDtypeStruct((hist * batch, PADW), jnp.float32),
        mesh=mesh,
        compiler_params=pltpu.CompilerParams(
            use_tc_tiling_on_sc=False, needs_layout_passes=False
        ),
        scratch_types=[
            pltpu.VMEM((hist, b_per_w), jnp.int32),
            pltpu.VMEM((GBUF, CHUNK, emb_dim), jnp.float32),
            pltpu.SemaphoreType.DMA((GBUF,)),
            pltpu.SemaphoreType.DMA((GBUF,)),
        ],
    )


def kernel(input, table):
    batch, hist = input.shape
    vocab, emb_dim = table.shape
    k = _build(batch, hist, emb_dim, vocab)
    scale = PADW // emb_dim
    inp_t = input.T.astype(jnp.int32) * jnp.int32(scale)
    tabp = jnp.pad(table, ((0, 0), (0, PADW - emb_dim)))
    tab4 = tabp.reshape(vocab * scale, emb_dim)
    inter = k(inp_t, tab4)
    out = _tc_transpose(inter, batch, hist, emb_dim)
    return out.transpose(2, 0, 1)


# speedup vs baseline: 3.2317x; 1.2807x over previous
import functools

import jax
import jax.numpy as jnp
from jax import lax
from jax.experimental import pallas as pl
from jax.experimental.pallas import tpu as pltpu
from jax.experimental.pallas import tpu_sc as plsc

CHUNK = 128
GBUF = 8
DEPTH = 4
PADW = 128


TC_COLS = 3968


def _tc_pad(table_t):
    d, v = table_t.shape

    def body(src_ref, dst_ref):
        t = src_ref[...].T
        dst_ref[...] = jnp.concatenate(
            [t, jnp.zeros((TC_COLS, PADW - d), jnp.float32)], axis=1
        )

    return pl.pallas_call(
        body,
        grid=((v + TC_COLS - 1) // TC_COLS,),
        in_specs=[pl.BlockSpec((d, TC_COLS), lambda j: (0, j))],
        out_specs=pl.BlockSpec((TC_COLS, PADW), lambda j: (j, 0)),
        out_shape=jax.ShapeDtypeStruct((v, PADW), jnp.float32),
    )(table_t)


def _tc_transpose(inter, batch, hist, emb_dim):
    def body(src_ref, dst_ref):
        dst_ref[...] = src_ref[:, :emb_dim].T[None]

    return pl.pallas_call(
        body,
        grid=(hist,),
        in_specs=[pl.BlockSpec((batch, PADW), lambda h: (h, 0))],
        out_specs=pl.BlockSpec((1, emb_dim, batch), lambda h: (h, 0, 0)),
        out_shape=jax.ShapeDtypeStruct((hist, emb_dim, batch), jnp.float32),
    )(inter)


@functools.cache
def _build(batch: int, hist: int, emb_dim: int, vocab: int):
    info = plsc.get_sparse_core_info()
    nc, ns = info.num_cores, info.num_subcores
    nw = nc * ns
    b_per_w = batch // nw
    assert batch % (nw * CHUNK) == 0
    n_sub = b_per_w // CHUNK
    n_chunks = hist * n_sub

    mesh = plsc.VectorSubcoreMesh(core_axis_name="c", subcore_axis_name="s")

    def body(idx_hbm, tab_hbm, inter_hbm, idx_v, buf_v, gsem, wsem):
        wid = lax.axis_index("s") * nc + lax.axis_index("c")
        b0 = wid * b_per_w
        pltpu.sync_copy(idx_hbm.at[:, pl.ds(b0, b_per_w)], idx_v)

        def gather(j, gb):
            h = j // n_sub
            c = lax.rem(j, n_sub)
            return pltpu.make_async_copy(
                tab_hbm.at[idx_v.at[h, pl.ds(c * CHUNK, CHUNK)]],
                buf_v.at[gb],
                gsem.at[gb],
            )

        def write(j, gb):
            h = j // n_sub
            c = lax.rem(j, n_sub)
            row0 = h * batch + b0 + c * CHUNK
            return pltpu.make_async_copy(
                buf_v.at[gb],
                inter_hbm.at[pl.ds(row0, CHUNK), pl.ds(0, emb_dim)],
                wsem.at[gb],
            )

        for j in range(DEPTH):
            gather(j, j).start()

        def step(j, carry):
            gb = lax.rem(j, jnp.int32(GBUF))
            gbn = lax.rem(j + DEPTH, jnp.int32(GBUF))

            @pl.when(j + DEPTH < n_chunks)
            def _refill():
                @pl.when(j >= GBUF - DEPTH)
                def _drain():
                    write(j + DEPTH - GBUF, gbn).wait()

                gather(j + DEPTH, gbn).start()

            gather(j, gb).wait()
            write(j, gb).start()
            return carry

        lax.fori_loop(0, n_chunks, step, 0, unroll=False)
        for j in range(n_chunks - GBUF, n_chunks):
            write(j, j % GBUF).wait()

    return pl.kernel(
        body,
        out_type=jax.ShapeDtypeStruct((hist * batch, PADW), jnp.float32),
        mesh=mesh,
        compiler_params=pltpu.CompilerParams(
            use_tc_tiling_on_sc=False, needs_layout_passes=False
        ),
        scratch_types=[
            pltpu.VMEM((hist, b_per_w), jnp.int32),
            pltpu.VMEM((GBUF, CHUNK, emb_dim), jnp.float32),
            pltpu.SemaphoreType.DMA((GBUF,)),
            pltpu.SemaphoreType.DMA((GBUF,)),
        ],
    )


def kernel(input, table):
    batch, hist = input.shape
    vocab, emb_dim = table.shape
    k = _build(batch, hist, emb_dim, vocab)
    scale = PADW // emb_dim
    inp_t = input.T.astype(jnp.int32) * jnp.int32(scale)
    tabp = _tc_pad(table.T)
    tab4 = tabp.reshape(vocab * scale, emb_dim)
    inter = k(inp_t, tab4)
    out = _tc_transpose(inter, batch, hist, emb_dim)
    return out.transpose(2, 0, 1)
